# baseline (speedup 1.0000x reference)
import functools

import numpy as np
import jax
import jax.numpy as jnp
from jax.experimental import pallas as pl
from jax.experimental.pallas import tpu as pltpu

_TD = 16
_NB = 22
_MAXD = 40.0
_N = 2048
_PP = 8
_NM = _NB - 2
_BM = _N // _PP
_KD = _PP * _NM
_LD = _PP * _TD

_BW = np.float64(np.float32(_MAXD / (_NB - 1)))
_THR = np.array(
    [float((np.float64(np.float32(m) * _BW)) ** 2 - 1e-8) for m in range(1, _NB - 1)],
    dtype=np.float32,
)


def _enc_body(ci_ref, xr_ref, yr_ref, zr_ref, e8_ref, thr_ref, dk_ref, tb_ref,
              o_ref):
    i = pl.program_id(0)
    xi = ci_ref[0, i]
    yi = ci_ref[1, i]
    zi = ci_ref[2, i]
    dx = xr_ref[...] - xi
    dy = yr_ref[...] - yi
    dz = zr_ref[...] - zi
    u8 = dx * dx + dy * dy + dz * dz
    u = jnp.dot(u8, e8_ref[...], preferred_element_type=jnp.float32)
    m = (u > thr_ref[...]).astype(jnp.float32)
    o_ref[...] = tb_ref[...] + jnp.dot(
        m, dk_ref[...], preferred_element_type=jnp.float32)


def _encode(xr, yr, zr, coords, e8, thrp, dk, tb):
    return pl.pallas_call(
        _enc_body,
        grid=(_N,),
        in_specs=[
            pl.BlockSpec(memory_space=pltpu.SMEM),
            pl.BlockSpec((_BM, _PP), lambda i: (0, 0)),
            pl.BlockSpec((_BM, _PP), lambda i: (0, 0)),
            pl.BlockSpec((_BM, _PP), lambda i: (0, 0)),
            pl.BlockSpec((_PP, _KD), lambda i: (0, 0)),
            pl.BlockSpec((1, _KD), lambda i: (0, 0)),
            pl.BlockSpec((_KD, _LD), lambda i: (0, 0)),
            pl.BlockSpec((1, _LD), lambda i: (0, 0)),
        ],
        out_specs=pl.BlockSpec((_BM, _LD), lambda i: (i, 0)),
        out_shape=jax.ShapeDtypeStruct((_N * _BM, _LD), jnp.float32),
    )(coords, xr, yr, zr, e8, thrp, dk, tb)


def kernel(coords, W, b):
    T = W.T + b[None, :]
    D = T[2:] - T[1:-1]
    dk = jnp.zeros((_KD, _LD), jnp.float32)
    for c in range(_PP):
        dk = dk.at[c * _NM:(c + 1) * _NM, c * _TD:(c + 1) * _TD].set(D)
    e8 = jnp.zeros((_PP, _KD), jnp.float32)
    for c in range(_PP):
        e8 = e8.at[c, c * _NM:(c + 1) * _NM].set(1.0)
    thrp = jnp.tile(jnp.asarray(_THR), (_PP,))[None, :]
    tb = jnp.tile(T[1], (_PP,))[None, :]
    xr = coords[:, 0].reshape(_BM, _PP)
    yr = coords[:, 1].reshape(_BM, _PP)
    zr = coords[:, 2].reshape(_BM, _PP)
    out = _encode(xr, yr, zr, coords.T, e8, thrp, dk, tb)
    return out.reshape(_N, _N, _TD)
